# baseline (speedup 1.0000x reference)
import functools

import jax
import jax.numpy as jnp
from jax import lax
from jax.experimental import pallas as pl
from jax.experimental.pallas import tpu as pltpu
from jax.experimental.pallas import tpu_sc as plsc

H = 4096
W = 4096
N = 4194304
NPIX = H * W
NROW = NPIX // 2

_NC = 2
_NS = 16
_NW = _NC * _NS
_NPW = N // _NW
_C = 2048
_NIT = _NPW // _C
_VR = _C // 16

_BUILD_LO = W // 2
_BUILD_COUNT = NROW - 2 * _BUILD_LO
_RPW = _BUILD_COUNT // _NW


_BB = 2048
_BNB = _RPW // _BB
_BREM = _RPW - _BNB * _BB


def _build_body(s0_hbm, s1_hbm, e2_hbm, t_hbm, t_v, sem):
    wid = lax.axis_index("s") * _NC + lax.axis_index("c")
    wlo = wid * _RPW

    def block(m, nrows, tv):
        pltpu.async_copy(s0_hbm.at[pl.ds(m, nrows)],
                         tv.at[:, pl.ds(0, 2)], sem)
        pltpu.async_copy(s1_hbm.at[pl.ds(m, nrows)],
                         tv.at[:, pl.ds(2, 2)], sem)
        pltpu.async_copy(e2_hbm.at[pl.ds(m + _BUILD_LO - W // 2, nrows)],
                         tv.at[:, pl.ds(4, 2)], sem)
        cp = pltpu.async_copy(e2_hbm.at[pl.ds(m + _BUILD_LO + W // 2, nrows)],
                              tv.at[:, pl.ds(6, 2)], sem)
        cp.wait()
        cp.wait()
        cp.wait()
        cp.wait()
        pltpu.sync_copy(tv, t_hbm.at[pl.ds(m + _BUILD_LO, nrows)])

    def blk(b, c):
        block(wlo + b * _BB, _BB, t_v)
        return c

    lax.fori_loop(0, _BNB, blk, None)
    if _BREM:
        block(wlo + _BNB * _BB, _BREM, t_v.at[pl.ds(0, _BREM)])


_sc_build = functools.partial(
    pl.kernel,
    mesh=plsc.VectorSubcoreMesh(core_axis_name="c", subcore_axis_name="s"),
    out_type=jax.ShapeDtypeStruct((NROW, 8), jnp.float32),
    scratch_types=[
        pltpu.VMEM((_BB, 8), jnp.float32),
        pltpu.SemaphoreType.DMA,
    ],
    compiler_params=pltpu.CompilerParams(use_tc_tiling_on_sc=False, needs_layout_passes=False),
)(_build_body)


def _gather_body(xx_hbm, xy_hbm, t_hbm, out_hbm,
                 xs_v, ys_v, idx0_v, idx1_v, g0_v, g1_v, o_v, sem0, sem1):
    wid = lax.axis_index("s") * _NC + lax.axis_index("c")
    wbase = wid * _NPW
    sems = (sem0, sem1)
    idxs = (idx0_v, idx1_v)
    gs = (g0_v, g1_v)
    iota16 = lax.iota(jnp.int32, 16)

    def fire(i, slot):
        base = wbase + i * _C
        xsb, ysb, idxb = xs_v.at[slot], ys_v.at[slot], idxs[slot]
        pltpu.sync_copy(xx_hbm.at[pl.ds(base, _C)], xsb)
        pltpu.sync_copy(xy_hbm.at[pl.ds(base, _C)], ysb)

        def build(j, c):
            lane = j * 16
            sx = xsb[pl.ds(lane, 16)] * 2048.0 + 2048.0
            sy = ysb[pl.ds(lane, 16)] * 2048.0 + 2048.0
            ixc = jnp.clip(sx.astype(jnp.int32), 1, W - 2)
            iyc = jnp.clip(sy.astype(jnp.int32), 1, H - 2)
            flat = iyc * W + ixc
            idxb[pl.ds(lane, 16)] = lax.shift_right_logical(flat, 1)
            return c

        lax.fori_loop(0, _VR, build, None)
        pltpu.async_copy(t_hbm.at[idxb], gs[slot], sems[slot])

    def drain(i, slot):
        base = wbase + i * _C
        xsb, ysb = xs_v.at[slot], ys_v.at[slot]
        gb = gs[slot]
        pltpu.make_async_copy(t_hbm.at[idxs[slot]], gb, sems[slot]).wait()

        def combine(j, c):
            lane = j * 16
            sx = xsb[pl.ds(lane, 16)] * 2048.0 + 2048.0
            sy = ysb[pl.ds(lane, 16)] * 2048.0 + 2048.0
            ix = sx.astype(jnp.int32)
            iy = sy.astype(jnp.int32)
            fx = sx - ix.astype(jnp.float32)
            fy = sy - iy.astype(jnp.float32)
            ixc = jnp.clip(ix, 1, W - 2)
            iyc = jnp.clip(iy, 1, H - 2)
            flat = iyc * W + ixc
            jj = jnp.bitwise_and(flat, 1)
            pt = lane + iota16
            exm = plsc.load_gather(gb, [pt, jj])
            e0 = plsc.load_gather(gb, [pt, jj + 1])
            exp_ = plsc.load_gather(gb, [pt, jj + 2])
            eym = plsc.load_gather(gb, [pt, jj + 4])
            eyp = plsc.load_gather(gb, [pt, jj + 6])
            dedx = 0.5 * (exp_ - exm)
            dedy = 0.5 * (eyp - eym)
            zero = jnp.float32(0.0)
            dx = jnp.maximum(jnp.maximum(-sx, zero),
                             jnp.maximum(sx - (W - 1), zero)) * (1.0 / 2048.0)
            dy = jnp.maximum(jnp.maximum(-sy, zero),
                             jnp.maximum(sy - (H - 1), zero)) * (1.0 / 2048.0)
            pen = dx * dx + dy * dy
            grad = fx * dedx + fy * dedy
            o_v[pl.ds(lane, 16)] = e0 + jnp.where(pen < 1e-6, grad, zero) + pen
            return c

        lax.fori_loop(0, _VR, combine, None)
        pltpu.sync_copy(o_v, out_hbm.at[pl.ds(base, _C)])

    fire(0, 0)

    def outer(k, carry):
        i = 2 * k
        fire(i + 1, 1)
        drain(i, 0)
        fire(i + 2, 0)
        drain(i + 1, 1)
        return carry

    lax.fori_loop(0, _NIT // 2 - 1, outer, None)
    fire(_NIT - 1, 1)
    drain(_NIT - 2, 0)
    drain(_NIT - 1, 1)


_sc_gather = functools.partial(
    pl.kernel,
    mesh=plsc.VectorSubcoreMesh(core_axis_name="c", subcore_axis_name="s"),
    out_type=jax.ShapeDtypeStruct((N,), jnp.float32),
    scratch_types=[
        pltpu.VMEM((2, _C), jnp.float32),
        pltpu.VMEM((2, _C), jnp.float32),
        pltpu.VMEM((_C,), jnp.int32),
        pltpu.VMEM((_C,), jnp.int32),
        pltpu.VMEM((_C, 8), jnp.float32),
        pltpu.VMEM((_C, 8), jnp.float32),
        pltpu.VMEM((_C,), jnp.float32),
        pltpu.SemaphoreType.DMA,
        pltpu.SemaphoreType.DMA,
    ],
    compiler_params=pltpu.CompilerParams(use_tc_tiling_on_sc=False, needs_layout_passes=False),
)(_gather_body)


def kernel(X, pixel_energy):
    e = pixel_energy.reshape(-1)
    pad = jnp.zeros((W + 2,), jnp.float32)
    ep = jnp.concatenate([pad, e, pad])
    off = W + 2
    cols = [ep[off + s: off + s + 2 * NROW: 2]
            for s in (-1, 0, 1, 2, -W, -W + 1, W, W + 1)]
    t = jnp.stack(cols, axis=1)
    xx = X[:, 0]
    xy = X[:, 1]
    out = _sc_gather(xx, xy, t)
    return out[:, None]


# speedup vs baseline: 11.4845x; 11.4845x over previous
import functools

import jax
import jax.numpy as jnp
from jax import lax
from jax.experimental import pallas as pl
from jax.experimental.pallas import tpu as pltpu
from jax.experimental.pallas import tpu_sc as plsc

H = 4096
W = 4096
N = 4194304
NPIX = H * W
NROW = NPIX // 2

_NC = 2
_NS = 16
_NW = _NC * _NS
_NPW = N // _NW
_C = 2048
_NIT = _NPW // _C
_VR = _C // 16

_BUILD_LO = W // 2
_BUILD_COUNT = NROW - 2 * _BUILD_LO
_RPW = _BUILD_COUNT // _NW


_BB = 2048
_BNB = _RPW // _BB
_BREM = _RPW - _BNB * _BB


def _build_body(e_hbm, t_hbm, xw_v, ym_v, yp_v, t_v, sem):
    wid = lax.axis_index("s") * _NC + lax.axis_index("c")
    wlo = wid * _RPW
    iota16 = lax.iota(jnp.int32, 16)

    def block(m, nrows, tv):
        g = 2 * (_BUILD_LO + m)
        cps = [
            pltpu.async_copy(
                e_hbm.at[pl.ds(g - 8, 2 * nrows + 16)],
                xw_v if nrows == _BB else xw_v.at[pl.ds(0, 2 * nrows + 16)],
                sem),
            pltpu.async_copy(
                e_hbm.at[pl.ds(g - W, 2 * nrows)],
                ym_v if nrows == _BB else ym_v.at[pl.ds(0, 2 * nrows)], sem),
            pltpu.async_copy(
                e_hbm.at[pl.ds(g + W, 2 * nrows)],
                yp_v if nrows == _BB else yp_v.at[pl.ds(0, 2 * nrows)], sem),
        ]
        for cp in cps:
            cp.wait()

        def grp(v, c2):
            lvec = v * 16 + iota16
            l2 = lvec + lvec
            for c in range(4):
                val = plsc.load_gather(xw_v, [l2 + (7 + c)])
                plsc.store_scatter(tv, [lvec, jnp.full((16,), c, jnp.int32)], val)
            for c, src in ((4, ym_v), (5, ym_v), (6, yp_v), (7, yp_v)):
                val = plsc.load_gather(src, [l2 + (c & 1)])
                plsc.store_scatter(tv, [lvec, jnp.full((16,), c, jnp.int32)], val)
            return c2

        lax.fori_loop(0, nrows // 16, grp, None)
        pltpu.sync_copy(tv, t_hbm.at[pl.ds(_BUILD_LO + m, nrows)])

    def blk(b, c):
        block(wlo + b * _BB, _BB, t_v)
        return c

    lax.fori_loop(0, _BNB, blk, None)
    if _BREM:
        block(wlo + _BNB * _BB, _BREM, t_v.at[pl.ds(0, _BREM)])


_sc_build = functools.partial(
    pl.kernel,
    mesh=plsc.VectorSubcoreMesh(core_axis_name="c", subcore_axis_name="s"),
    out_type=jax.ShapeDtypeStruct((NROW, 8), jnp.float32),
    scratch_types=[
        pltpu.VMEM((2 * _BB + 16,), jnp.float32),
        pltpu.VMEM((2 * _BB,), jnp.float32),
        pltpu.VMEM((2 * _BB,), jnp.float32),
        pltpu.VMEM((_BB, 8), jnp.float32),
        pltpu.SemaphoreType.DMA,
    ],
    compiler_params=pltpu.CompilerParams(use_tc_tiling_on_sc=False, needs_layout_passes=False),
)(_build_body)


def _gather_body(xx_hbm, xy_hbm, t_hbm, out_hbm,
                 xs_v, ys_v, idx0_v, idx1_v, g0_v, g1_v, o_v, sem0, sem1):
    wid = lax.axis_index("s") * _NC + lax.axis_index("c")
    wbase = wid * _NPW
    sems = (sem0, sem1)
    idxs = (idx0_v, idx1_v)
    gs = (g0_v, g1_v)
    iota16 = lax.iota(jnp.int32, 16)

    def fire(i, slot):
        base = wbase + i * _C
        xsb, ysb, idxb = xs_v.at[slot], ys_v.at[slot], idxs[slot]
        pltpu.sync_copy(xx_hbm.at[pl.ds(base, _C)], xsb)
        pltpu.sync_copy(xy_hbm.at[pl.ds(base, _C)], ysb)

        def build(j, c):
            lane = j * 16
            sx = xsb[pl.ds(lane, 16)] * 2048.0 + 2048.0
            sy = ysb[pl.ds(lane, 16)] * 2048.0 + 2048.0
            ixc = jnp.clip(sx.astype(jnp.int32), 1, W - 2)
            iyc = jnp.clip(sy.astype(jnp.int32), 1, H - 2)
            flat = iyc * W + ixc
            idxb[pl.ds(lane, 16)] = lax.shift_right_logical(flat, 1)
            return c

        lax.fori_loop(0, _VR, build, None)
        pltpu.async_copy(t_hbm.at[idxb], gs[slot], sems[slot])

    def drain(i, slot):
        base = wbase + i * _C
        xsb, ysb = xs_v.at[slot], ys_v.at[slot]
        gb = gs[slot]
        pltpu.make_async_copy(t_hbm.at[idxs[slot]], gb, sems[slot]).wait()

        def combine(j, c):
            lane = j * 16
            sx = xsb[pl.ds(lane, 16)] * 2048.0 + 2048.0
            sy = ysb[pl.ds(lane, 16)] * 2048.0 + 2048.0
            ix = sx.astype(jnp.int32)
            iy = sy.astype(jnp.int32)
            fx = sx - ix.astype(jnp.float32)
            fy = sy - iy.astype(jnp.float32)
            ixc = jnp.clip(ix, 1, W - 2)
            iyc = jnp.clip(iy, 1, H - 2)
            flat = iyc * W + ixc
            jj = jnp.bitwise_and(flat, 1)
            pt = lane + iota16
            exm = plsc.load_gather(gb, [pt, jj])
            e0 = plsc.load_gather(gb, [pt, jj + 1])
            exp_ = plsc.load_gather(gb, [pt, jj + 2])
            eym = plsc.load_gather(gb, [pt, jj + 4])
            eyp = plsc.load_gather(gb, [pt, jj + 6])
            dedx = 0.5 * (exp_ - exm)
            dedy = 0.5 * (eyp - eym)
            zero = jnp.float32(0.0)
            dx = jnp.maximum(jnp.maximum(-sx, zero),
                             jnp.maximum(sx - (W - 1), zero)) * (1.0 / 2048.0)
            dy = jnp.maximum(jnp.maximum(-sy, zero),
                             jnp.maximum(sy - (H - 1), zero)) * (1.0 / 2048.0)
            pen = dx * dx + dy * dy
            grad = fx * dedx + fy * dedy
            o_v[pl.ds(lane, 16)] = e0 + jnp.where(pen < 1e-6, grad, zero) + pen
            return c

        lax.fori_loop(0, _VR, combine, None)
        pltpu.sync_copy(o_v, out_hbm.at[pl.ds(base, _C)])

    fire(0, 0)

    def outer(k, carry):
        i = 2 * k
        fire(i + 1, 1)
        drain(i, 0)
        fire(i + 2, 0)
        drain(i + 1, 1)
        return carry

    lax.fori_loop(0, _NIT // 2 - 1, outer, None)
    fire(_NIT - 1, 1)
    drain(_NIT - 2, 0)
    drain(_NIT - 1, 1)


_sc_gather = functools.partial(
    pl.kernel,
    mesh=plsc.VectorSubcoreMesh(core_axis_name="c", subcore_axis_name="s"),
    out_type=jax.ShapeDtypeStruct((N,), jnp.float32),
    scratch_types=[
        pltpu.VMEM((2, _C), jnp.float32),
        pltpu.VMEM((2, _C), jnp.float32),
        pltpu.VMEM((_C,), jnp.int32),
        pltpu.VMEM((_C,), jnp.int32),
        pltpu.VMEM((_C, 8), jnp.float32),
        pltpu.VMEM((_C, 8), jnp.float32),
        pltpu.VMEM((_C,), jnp.float32),
        pltpu.SemaphoreType.DMA,
        pltpu.SemaphoreType.DMA,
    ],
    compiler_params=pltpu.CompilerParams(use_tc_tiling_on_sc=False, needs_layout_passes=False),
)(_gather_body)


def kernel(X, pixel_energy):
    e = pixel_energy.reshape(-1)
    t = _sc_build(e)
    xx = X[:, 0]
    xy = X[:, 1]
    out = _sc_gather(xx, xy, t)
    return out[:, None]
